# baseline (speedup 1.0000x reference)
import math
import functools

import jax
import jax.numpy as jnp
from jax import lax
from jax.experimental import pallas as pl
from jax.experimental.pallas import tpu as pltpu


def _logits_body(x_ref, gw_ref, out_ref):
    out_ref[...] = lax.dot_general(
        gw_ref[...], x_ref[...], (((1,), (1,)), ((), ())),
        preferred_element_type=jnp.float32,
        precision=lax.Precision.HIGHEST)


def _router_logits(xf, gate_w):
    T, H = xf.shape
    E = gate_w.shape[0]
    return pl.pallas_call(
        _logits_body,
        out_shape=jax.ShapeDtypeStruct((E, T), jnp.float32),
    )(xf, gate_w)


def _ffn_body(nit, score_ref, xg_ref, w1_ref, w3_ref, w2_ref, out_ref,
              acc_ref):
    it = pl.program_id(1)

    @pl.when(it == 0)
    def _init():
        acc_ref[...] = jnp.zeros_like(acc_ref)

    xb = xg_ref[0]
    w1 = w1_ref[0]
    w3 = w3_ref[0]
    w2 = w2_ref[0]
    a = lax.dot_general(xb, w1, (((1,), (1,)), ((), ())),
                        preferred_element_type=jnp.float32)
    b = lax.dot_general(xb, w3, (((1,), (1,)), ((), ())),
                        preferred_element_type=jnp.float32)
    h = (a * jax.nn.sigmoid(a) * b).astype(jnp.bfloat16)
    acc_ref[...] += lax.dot_general(h, w2, (((1,), (1,)), ((), ())),
                                    preferred_element_type=jnp.float32)

    @pl.when(it == nit - 1)
    def _fin():
        s = score_ref[0]
        m = jnp.max(s, axis=-1, keepdims=True)
        ex = jnp.exp(s - m)
        w = ex / jnp.sum(ex, axis=-1, keepdims=True)
        out_ref[0] = acc_ref[...] * w.reshape(-1, 1)


def _ffn(selscore, xg, w1b, w3b, w2b):
    E, CAP, H = xg.shape
    I = w1b.shape[1]
    TI = min(1024, I)
    NIT = I // TI
    grid = (E, NIT)
    return pl.pallas_call(
        functools.partial(_ffn_body, NIT),
        grid=grid,
        in_specs=[
            pl.BlockSpec((1, 1, CAP), lambda e, i: (e, 0, 0)),
            pl.BlockSpec((1, CAP, H), lambda e, i: (e, 0, 0)),
            pl.BlockSpec((1, TI, H), lambda e, i: (e, i, 0)),
            pl.BlockSpec((1, TI, H), lambda e, i: (e, i, 0)),
            pl.BlockSpec((1, H, TI), lambda e, i: (e, 0, i)),
        ],
        out_specs=pl.BlockSpec((1, CAP, H), lambda e, i: (e, 0, 0)),
        out_shape=jax.ShapeDtypeStruct((E, CAP, H), jnp.float32),
        scratch_shapes=[pltpu.VMEM((CAP, H), jnp.float32)],
    )(selscore.reshape(E, 1, CAP), xg, w1b, w3b, w2b)


def _combine_body(nexp, sel_ref, eo_ref, out_ref):
    e = pl.program_id(0)
    T = out_ref.shape[0]
    CAP = sel_ref.shape[2]

    @pl.when(e == 0)
    def _init():
        out_ref[...] = jnp.zeros_like(out_ref)

    sel = sel_ref[0]
    tcol = lax.broadcasted_iota(jnp.int32, (T, CAP), 0)
    P = (tcol == sel).astype(jnp.bfloat16)
    out_ref[...] += lax.dot_general(
        P, eo_ref[0], (((1,), (0,)), ((), ())),
        preferred_element_type=jnp.float32)


def _combine(sel, eo_bf, T):
    E, CAP, H = eo_bf.shape
    return pl.pallas_call(
        functools.partial(_combine_body, E),
        grid=(E,),
        in_specs=[
            pl.BlockSpec((1, 1, CAP), lambda e: (e, 0, 0)),
            pl.BlockSpec((1, CAP, H), lambda e: (e, 0, 0)),
        ],
        out_specs=pl.BlockSpec((T, H), lambda e: (0, 0)),
        out_shape=jax.ShapeDtypeStruct((T, H), jnp.float32),
    )(sel.reshape(E, 1, CAP), eo_bf)


def kernel(x, gate_w, w1, w2, w3):
    B, S, H = x.shape
    E = gate_w.shape[0]
    T = B * S
    cap = min(int(math.ceil(T / E * 1.25)), T)

    xf = x.reshape(T, H)
    logits = (xf @ gate_w.T).T

    selscore, sel = lax.top_k(logits, cap)
    xg = jnp.take(xf, sel.reshape(-1), axis=0).reshape(E, cap, H)

    eo = _ffn(selscore, xg.astype(jnp.bfloat16),
              w1.astype(jnp.bfloat16), w3.astype(jnp.bfloat16),
              w2.astype(jnp.bfloat16))

    out = _combine(sel, eo.astype(jnp.bfloat16), T)
    return out.reshape(B, S, H), jnp.array(0.0, dtype=jnp.float32)


# speedup vs baseline: 1.8859x; 1.8859x over previous
import math
import functools

import jax
import jax.numpy as jnp
from jax import lax
from jax.experimental import pallas as pl
from jax.experimental.pallas import tpu as pltpu


def _logits_body(x_ref, gw_ref, out_ref):
    out_ref[...] = lax.dot_general(
        gw_ref[...], x_ref[...], (((1,), (1,)), ((), ())),
        preferred_element_type=jnp.float32,
        precision=lax.Precision.HIGHEST)


def _router_logits(xf, gate_w):
    T, H = xf.shape
    E = gate_w.shape[0]
    return pl.pallas_call(
        _logits_body,
        out_shape=jax.ShapeDtypeStruct((E, T), jnp.float32),
    )(xf, gate_w)


def _ffn_body(nit, score_ref, xg_ref, w1_ref, w3_ref, w2_ref, out_ref,
              acc_ref):
    it = pl.program_id(1)

    @pl.when(it == 0)
    def _init():
        acc_ref[...] = jnp.zeros_like(acc_ref)

    xb = xg_ref[0]
    w1 = w1_ref[0].astype(jnp.bfloat16)
    w3 = w3_ref[0].astype(jnp.bfloat16)
    w2 = w2_ref[0].astype(jnp.bfloat16)
    a = lax.dot_general(xb, w1, (((1,), (1,)), ((), ())),
                        preferred_element_type=jnp.float32)
    b = lax.dot_general(xb, w3, (((1,), (1,)), ((), ())),
                        preferred_element_type=jnp.float32)
    h = (a * jax.nn.sigmoid(a) * b).astype(jnp.bfloat16)
    acc_ref[...] += lax.dot_general(h, w2, (((1,), (1,)), ((), ())),
                                    preferred_element_type=jnp.float32)

    @pl.when(it == nit - 1)
    def _fin():
        s = score_ref[0]
        m = jnp.max(s, axis=-1, keepdims=True)
        ex = jnp.exp(s - m)
        w = ex / jnp.sum(ex, axis=-1, keepdims=True)
        out_ref[0] = (acc_ref[...] * w.reshape(-1, 1)).astype(jnp.bfloat16)


def _ffn(selscore, xg, w1b, w3b, w2b):
    E, CAP, H = xg.shape
    I = w1b.shape[1]
    TI = min(512, I)
    NIT = I // TI
    grid = (E, NIT)
    return pl.pallas_call(
        functools.partial(_ffn_body, NIT),
        grid=grid,
        in_specs=[
            pl.BlockSpec((1, 1, CAP), lambda e, i: (e, 0, 0)),
            pl.BlockSpec((1, CAP, H), lambda e, i: (e, 0, 0)),
            pl.BlockSpec((1, TI, H), lambda e, i: (e, i, 0)),
            pl.BlockSpec((1, TI, H), lambda e, i: (e, i, 0)),
            pl.BlockSpec((1, H, TI), lambda e, i: (e, 0, i)),
        ],
        out_specs=pl.BlockSpec((1, CAP, H), lambda e, i: (e, 0, 0)),
        out_shape=jax.ShapeDtypeStruct((E, CAP, H), jnp.bfloat16),
        scratch_shapes=[pltpu.VMEM((CAP, H), jnp.float32)],
    )(selscore.reshape(E, 1, CAP), xg, w1b, w3b, w2b)


def _combine_body(nexp, sel_ref, eo_ref, out_ref):
    e = pl.program_id(0)
    T = out_ref.shape[0]
    CAP = sel_ref.shape[2]

    @pl.when(e == 0)
    def _init():
        out_ref[...] = jnp.zeros_like(out_ref)

    sel = sel_ref[0]
    tcol = lax.broadcasted_iota(jnp.int32, (T, CAP), 0)
    P = (tcol == sel).astype(jnp.bfloat16)
    out_ref[...] += lax.dot_general(
        P, eo_ref[0], (((1,), (0,)), ((), ())),
        preferred_element_type=jnp.float32)


def _combine(sel, eo_bf, T):
    E, CAP, H = eo_bf.shape
    return pl.pallas_call(
        functools.partial(_combine_body, E),
        grid=(E,),
        in_specs=[
            pl.BlockSpec((1, 1, CAP), lambda e: (e, 0, 0)),
            pl.BlockSpec((1, CAP, H), lambda e: (e, 0, 0)),
        ],
        out_specs=pl.BlockSpec((T, H), lambda e: (0, 0)),
        out_shape=jax.ShapeDtypeStruct((T, H), jnp.float32),
    )(sel.reshape(E, 1, CAP), eo_bf)


def kernel(x, gate_w, w1, w2, w3):
    B, S, H = x.shape
    E = gate_w.shape[0]
    T = B * S
    cap = min(int(math.ceil(T / E * 1.25)), T)

    xf = x.reshape(T, H)
    logits = (xf @ gate_w.T).T

    selscore, sel = lax.top_k(logits, cap)
    xg = jnp.take(xf, sel.reshape(-1), axis=0).reshape(E, cap, H)

    eo = _ffn(selscore, xg.astype(jnp.bfloat16),
              w1, w3, w2)

    out = _combine(sel, eo, T)
    return out.reshape(B, S, H), jnp.array(0.0, dtype=jnp.float32)
